# baseline (speedup 1.0000x reference)
import functools

import jax
import jax.numpy as jnp
from jax.experimental import pallas as pl
from jax.experimental.pallas import tpu as pltpu

_W = 2048
_R = 4
_QC = _W // _R


def _attn_body(xq_ref, xs_ref, xd_ref, wq_ref, wk_ref, wv_ref, out_ref):
    scale = 1.0 / 16.0
    xq = xq_ref[0, 0]
    xs = xs_ref[0]
    xd = xd_ref[0]
    wq = wq_ref[...]
    wk = wk_ref[...]
    wv = wv_ref[...]

    f32 = jnp.float32
    bf = jnp.bfloat16
    k = jnp.dot(xs, wk, preferred_element_type=f32).astype(bf)
    v = jnp.dot(xs, wv, preferred_element_type=f32).astype(bf)
    kd = jnp.dot(xd, wk, preferred_element_type=f32).astype(bf)
    vd = jnp.dot(xd, wv, preferred_element_type=f32).astype(bf)

    for i in range(_R):
        qi = jnp.dot(xq[:, i, :], wq, preferred_element_type=f32).astype(bf)
        s = jax.lax.dot_general(qi, k, (((1,), (1,)), ((), ())),
                                preferred_element_type=f32) * scale
        p = jnp.exp(s)
        den = jnp.sum(p, axis=1)
        u = jnp.dot(p.astype(bf), v, preferred_element_type=f32)
        if i == 0:
            sd = jax.lax.dot_general(qi, kd, (((1,), (1,)), ((), ())),
                                     preferred_element_type=f32) * scale
            pd = jnp.exp(sd)
            den = den + jnp.sum(pd, axis=1)
            u = u + jnp.dot(pd.astype(bf), vd, preferred_element_type=f32)
        out_ref[0, :, i, :] = u * (1.0 / den)[:, None]


def kernel(x, Wq, Wk, Wv):
    b, n, c = x.shape
    nseg = n // _W
    xb = x.astype(jnp.bfloat16)
    x4 = xb.reshape(b, nseg, _QC, _R, c)
    xr = xb.reshape(b * nseg, _W, c)
    xd4 = xb.reshape(b, n // _R, _R * c)
    wqb = Wq.astype(jnp.bfloat16)
    wkb = Wk.astype(jnp.bfloat16)
    wvb = Wv.astype(jnp.bfloat16)

    grid = (b, nseg)
    out4 = pl.pallas_call(
        _attn_body,
        grid=grid,
        in_specs=[
            pl.BlockSpec((1, 1, _QC, _R, c), lambda bi, si: (bi, si, 0, 0, 0)),
            pl.BlockSpec((1, _W, c), lambda bi, si: (bi * 4 + si, 0, 0)),
            pl.BlockSpec((1, n // _R, c), lambda bi, si: (bi, 0, 0)),
            pl.BlockSpec((c, c), lambda bi, si: (0, 0)),
            pl.BlockSpec((c, c), lambda bi, si: (0, 0)),
            pl.BlockSpec((c, c), lambda bi, si: (0, 0)),
        ],
        out_specs=pl.BlockSpec((1, _QC, _R, c), lambda bi, si: (bi, si, 0, 0)),
        out_shape=jax.ShapeDtypeStruct((b, n // _R, _R, c), jnp.float32),
    )(x4, xr, xd4, wqb, wkb, wvb)
    return out4.reshape(b, n, c)


# speedup vs baseline: 1.1143x; 1.1143x over previous
import jax
import jax.numpy as jnp
from jax.experimental import pallas as pl
from jax.experimental.pallas import tpu as pltpu

_W = 2048
_R = 4
_QC = _W // _R
_F32 = jnp.float32


def _attn_body(xq0_ref, xq1_ref, xq2_ref, xq3_ref, xs_ref, xd_ref,
               wq_ref, wk_ref, wv_ref, out_ref, kd_ref, vd_ref):
    si = pl.program_id(1)
    xs = xs_ref[0]
    wq = wq_ref[...]
    wk = wk_ref[...]
    wv = wv_ref[...]

    k = jnp.dot(xs, wk, preferred_element_type=_F32)
    v = jnp.dot(xs, wv, preferred_element_type=_F32)

    @pl.when(si == 0)
    def _project_dilated():
        xd = xd_ref[0]
        kd_ref[...] = jnp.dot(xd, wk, preferred_element_type=_F32)
        vd_ref[...] = jnp.dot(xd, wv, preferred_element_type=_F32)

    c = xs.shape[1]
    for i, xq_ref in enumerate((xq0_ref, xq1_ref, xq2_ref, xq3_ref)):
        qi = jnp.dot(xq_ref[0, 0], wq, preferred_element_type=_F32)
        s = jax.lax.dot_general(qi, k, (((1,), (1,)), ((), ())),
                                preferred_element_type=_F32)
        p = jnp.exp2(s)
        den = jnp.sum(p, axis=1)
        u = jnp.dot(p, v, preferred_element_type=_F32)
        if i == 0:
            sd = jax.lax.dot_general(qi, kd_ref[...], (((1,), (1,)), ((), ())),
                                     preferred_element_type=_F32)
            pd = jnp.exp2(sd)
            den = den + jnp.sum(pd, axis=1)
            u = u + jnp.dot(pd, vd_ref[...], preferred_element_type=_F32)
        out_ref[0, :, i * c:(i + 1) * c] = u * (1.0 / den)[:, None]


def kernel(x, Wq, Wk, Wv):
    b, n, c = x.shape
    nseg = n // _W
    nd = n // _R
    xq4 = x.reshape(b, nseg, _QC, _R * c)
    xr = x.reshape(b * nseg, _W, c)
    xdc = x.reshape(b, nd, _R * c)
    log2e = 1.4426950408889634
    wq_s = Wq * (log2e / jnp.sqrt(jnp.asarray(c, _F32)))

    def qspec(i):
        return pl.BlockSpec((1, 1, _QC, c), lambda bi, si, i=i: (bi, si, 0, i))

    out_c = pl.pallas_call(
        _attn_body,
        grid=(b, nseg),
        in_specs=[
            qspec(0), qspec(1), qspec(2), qspec(3),
            pl.BlockSpec((1, _W, c), lambda bi, si: (bi * 4 + si, 0, 0)),
            pl.BlockSpec((1, nd, c), lambda bi, si: (bi, 0, 0)),
            pl.BlockSpec((c, c), lambda bi, si: (0, 0)),
            pl.BlockSpec((c, c), lambda bi, si: (0, 0)),
            pl.BlockSpec((c, c), lambda bi, si: (0, 0)),
        ],
        out_specs=pl.BlockSpec((1, _QC, _R * c), lambda bi, si: (bi, si, 0)),
        out_shape=jax.ShapeDtypeStruct((b, nd, _R * c), _F32),
        scratch_shapes=[pltpu.VMEM((nd, c), _F32), pltpu.VMEM((nd, c), _F32)],
    )(xq4, xq4, xq4, xq4, xr, xdc, wq_s, Wk, Wv)
    return out_c.reshape(b, n, c)


# speedup vs baseline: 1.3003x; 1.1670x over previous
import jax
import jax.numpy as jnp
from jax.experimental import pallas as pl
from jax.experimental.pallas import tpu as pltpu

_W = 2048
_R = 4
_QC = _W // _R
_F32 = jnp.float32


def _attn_body(xq0_ref, xq1_ref, xq2_ref, xq3_ref, xs_ref, xd_ref,
               wq_ref, wk_ref, wv_ref, out_ref, kd_ref, vd_ref):
    si = pl.program_id(1)
    xs = xs_ref[0]
    wq = wq_ref[...]
    wk = wk_ref[...]
    wv = wv_ref[...]

    k = jnp.dot(xs, wk, preferred_element_type=_F32)
    v = jnp.dot(xs, wv, preferred_element_type=_F32)

    @pl.when(si == 0)
    def _project_dilated():
        xd = xd_ref[0]
        kd_ref[...] = jnp.dot(xd, wk, preferred_element_type=_F32)
        vd_ref[...] = jnp.dot(xd, wv, preferred_element_type=_F32)

    c = xs.shape[1]
    for i, xq_ref in enumerate((xq0_ref, xq1_ref, xq2_ref, xq3_ref)):
        qi = jnp.dot(xq_ref[0], wq, preferred_element_type=_F32)
        s = jax.lax.dot_general(qi, k, (((1,), (1,)), ((), ())),
                                preferred_element_type=_F32)
        p = jnp.exp2(s)
        den = jnp.sum(p, axis=1)
        u = jnp.dot(p, v, preferred_element_type=_F32)
        if i == 0:
            sd = jax.lax.dot_general(qi, kd_ref[...], (((1,), (1,)), ((), ())),
                                     preferred_element_type=_F32)
            pd = jnp.exp2(sd)
            den = den + jnp.sum(pd, axis=1)
            u = u + jnp.dot(pd, vd_ref[...], preferred_element_type=_F32)
        out_ref[0, :, i * c:(i + 1) * c] = u * (1.0 / den)[:, None]


def kernel(x, Wq, Wk, Wv):
    b, n, c = x.shape
    nseg = n // _W
    nd = n // _R
    x1024 = x.reshape(b, nd, _R * c)
    log2e = 1.4426950408889634
    wq_s = Wq * (log2e / jnp.sqrt(jnp.asarray(c, _F32)))

    def qspec(i):
        return pl.BlockSpec((1, _QC, c), lambda bi, si, i=i: (bi, si, i))

    out_c = pl.pallas_call(
        _attn_body,
        grid=(b, nseg),
        in_specs=[
            qspec(0), qspec(1), qspec(2), qspec(3),
            pl.BlockSpec((1, _W, c), lambda bi, si: (bi, si, 0)),
            pl.BlockSpec((1, nd, c), lambda bi, si: (bi, 0, 0)),
            pl.BlockSpec((c, c), lambda bi, si: (0, 0)),
            pl.BlockSpec((c, c), lambda bi, si: (0, 0)),
            pl.BlockSpec((c, c), lambda bi, si: (0, 0)),
        ],
        out_specs=pl.BlockSpec((1, _QC, _R * c), lambda bi, si: (bi, si, 0)),
        out_shape=jax.ShapeDtypeStruct((b, nd, _R * c), _F32),
        scratch_shapes=[pltpu.VMEM((nd, c), _F32), pltpu.VMEM((nd, c), _F32)],
    )(x1024, x1024, x1024, x1024, x, x1024, wq_s, Wk, Wv)
    return out_c.reshape(b, n, c)


# speedup vs baseline: 1.4363x; 1.1045x over previous
import jax
import jax.numpy as jnp
from jax.experimental import pallas as pl
from jax.experimental.pallas import tpu as pltpu

_W = 2048
_R = 4
_QC = _W // _R
_F32 = jnp.float32


def _attn_body(xq0_ref, xq1_ref, xq2_ref, xq3_ref, xs_ref, xd_ref,
               wq_ref, wk_ref, wv_ref, out_ref, kd_ref, vd_ref):
    si = pl.program_id(1)
    xs = xs_ref[0]
    wq = wq_ref[...]
    wk = wk_ref[...]
    wv = wv_ref[...]

    k = jnp.dot(xs, wk, preferred_element_type=_F32)
    v = jnp.dot(xs, wv, preferred_element_type=_F32)

    @pl.when(si == 0)
    def _project_dilated():
        xd = xd_ref[0]
        kd_ref[...] = jnp.dot(xd, wk, preferred_element_type=_F32)
        vd_ref[...] = jnp.dot(xd, wv, preferred_element_type=_F32)

    us = []
    for i, xq_ref in enumerate((xq0_ref, xq1_ref, xq2_ref, xq3_ref)):
        qi = jnp.dot(xq_ref[0], wq, preferred_element_type=_F32)
        s = jax.lax.dot_general(qi, k, (((1,), (1,)), ((), ())),
                                preferred_element_type=_F32)
        p = jnp.exp2(s)
        den = jnp.sum(p, axis=1)
        u = jnp.dot(p, v, preferred_element_type=_F32)
        if i == 0:
            sd = jax.lax.dot_general(qi, kd_ref[...], (((1,), (1,)), ((), ())),
                                     preferred_element_type=_F32)
            pd = jnp.exp2(sd)
            den = den + jnp.sum(pd, axis=1)
            u = u + jnp.dot(pd, vd_ref[...], preferred_element_type=_F32)
        us.append(u * (1.0 / den)[:, None])
    out_ref[0] = jnp.stack(us, axis=1).reshape(_W, xs.shape[1])


def kernel(x, Wq, Wk, Wv):
    b, n, c = x.shape
    nseg = n // _W
    nd = n // _R
    x1024 = x.reshape(b, nd, _R * c)
    log2e = 1.4426950408889634
    wq_s = Wq * (log2e / jnp.sqrt(jnp.asarray(c, _F32)))

    def qspec(i):
        return pl.BlockSpec((1, _QC, c), lambda bi, si, i=i: (bi, si, i))

    out = pl.pallas_call(
        _attn_body,
        grid=(b, nseg),
        in_specs=[
            qspec(0), qspec(1), qspec(2), qspec(3),
            pl.BlockSpec((1, _W, c), lambda bi, si: (bi, si, 0)),
            pl.BlockSpec((1, nd, c), lambda bi, si: (bi, 0, 0)),
            pl.BlockSpec((c, c), lambda bi, si: (0, 0)),
            pl.BlockSpec((c, c), lambda bi, si: (0, 0)),
            pl.BlockSpec((c, c), lambda bi, si: (0, 0)),
        ],
        out_specs=pl.BlockSpec((1, _W, c), lambda bi, si: (bi, si, 0)),
        out_shape=jax.ShapeDtypeStruct((b, n, c), _F32),
        scratch_shapes=[pltpu.VMEM((nd, c), _F32), pltpu.VMEM((nd, c), _F32)],
    )(x1024, x1024, x1024, x1024, x, x1024, wq_s, Wk, Wv)
    return out
